# Initial kernel scaffold; baseline (speedup 1.0000x reference)
import jax
import jax.numpy as jnp
from jax.experimental import pallas as pl


def kernel(x, x_lens, y, y_lens, W_enc, b_enc, emb, W_am, b_am, W_lm, b_lm):
    raise NotImplementedError("write your pallas kernel here")


# speedup vs baseline: 162.6775x; 162.6775x over previous
import jax
import jax.numpy as jnp
from jax import lax
from jax.experimental import pallas as pl
from jax.experimental.pallas import tpu as pltpu

N, T, F = 8, 512, 80
U = 128
U1 = U + 1
D_ENC = 512
D_DEC = 512
V = 500
NEG = -1e30
NDIAG = T + U1 - 1


def _dot(a, b):
    return lax.dot_general(
        a, b, (((1,), (0,)), ((), ())),
        preferred_element_type=jnp.float32,
        precision=lax.Precision.HIGHEST)


def _dot_nt(a, b):
    return lax.dot_general(
        a, b, (((1,), (1,)), ((), ())),
        preferred_element_type=jnp.float32,
        precision=lax.Precision.HIGHEST)


def _logaddexp(a, b):
    m = jnp.maximum(a, b)
    d = -jnp.abs(a - b)
    return m + jnp.log1p(jnp.exp(d))


def _rnnt_kernel(x_ref, xlen_ref, ylen_ref, sos_ref, ypad_ref,
                 wenc_ref, benc_ref, emb_ref, wam_ref, bam_ref,
                 wlm_ref, blm_ref, out_ref, sb_ref, sl_ref):
    f32 = jnp.float32
    x = x_ref[...]
    x_lens = xlen_ref[...]
    y_lens = ylen_ref[...]

    M = _dot(wenc_ref[...], wam_ref[...])
    c_am = _dot(benc_ref[...], wam_ref[...]) + bam_ref[...]
    am = _dot(x, M) + c_am
    amax = jnp.max(am, axis=1, keepdims=True)
    eam = jnp.exp(am - amax)

    P = _dot(emb_ref[...], wlm_ref[...]) + blm_ref[...]

    e0 = (lax.broadcasted_iota(jnp.int32, (1, V), 1) == 0).astype(f32)
    u_eye = (lax.broadcasted_iota(jnp.int32, (U1, U1), 0) ==
             lax.broadcasted_iota(jnp.int32, (U1, U1), 1)).astype(f32)
    ones_row = jnp.ones((1, U1), dtype=f32)
    uv_iota = lax.broadcasted_iota(jnp.int32, (U1, V), 1)

    blank_rows = []
    label_rows = []
    lm0_rows = []
    lmy_rows = []
    bmax_rows = []
    for n in range(N):
        sos_n = sos_ref[n]
        ypad_n = ypad_ref[n]
        soh_n = (sos_n == uv_iota).astype(f32)
        yoh_n = (ypad_n == uv_iota).astype(f32)
        lm_n = _dot(soh_n, P)
        bmax_n = jnp.max(lm_n)
        elm_n = jnp.exp(lm_n - bmax_n)

        am_n = am[n * T:(n + 1) * T]
        eam_n = eam[n * T:(n + 1) * T]
        amax_n = amax[n * T:(n + 1) * T]
        zlog_n = jnp.log(_dot_nt(eam_n, elm_n)) + amax_n
        am_y_n = _dot_nt(am_n, yoh_n)
        am0_n = am_n[:, 0:1]

        blank_rows.append((am0_n - zlog_n).reshape(T, 1, U1))
        label_rows.append((am_y_n - zlog_n).reshape(T, 1, U1))
        lm0_rows.append(_dot_nt(e0, lm_n))
        lmy_col = jnp.sum(lm_n * yoh_n, axis=1, keepdims=True)
        lmy_rows.append(_dot(ones_row, u_eye * lmy_col))
        bmax_rows.append(bmax_n.reshape(1, 1))

    blank = jnp.concatenate(blank_rows, axis=1)
    label = jnp.concatenate(label_rows, axis=1)
    lm0 = jnp.concatenate(lm0_rows, axis=0)
    lm_y = jnp.concatenate(lmy_rows, axis=0)
    bmax = jnp.concatenate(bmax_rows, axis=0)

    pad = jnp.full((NDIAG - T, N, U1), NEG, dtype=f32)
    sb = jnp.concatenate([blank, pad], axis=0)
    sl = jnp.concatenate([label, pad], axis=0)
    lane_u = lax.broadcasted_iota(jnp.int32, (1, 1, U1), 2)
    for bit in range(8):
        sh = 1 << bit
        mask = (lane_u & sh) != 0
        fill = jnp.full((sh, N, U1), NEG, dtype=f32)
        sb = jnp.where(mask, jnp.concatenate([fill, sb[:-sh]], axis=0), sb)
        sl = jnp.where(mask, jnp.concatenate([fill, sl[:-sh]], axis=0), sl)

    u_iota = lax.broadcasted_iota(jnp.int32, (N, U1), 1)
    valid_u = u_iota < y_lens
    sb = sb + (lm0 - bmax)[None]
    sl = jnp.where(valid_u[None], sl + (lm_y - bmax)[None], NEG)
    sb_ref[...] = sb
    sl_ref[...] = sl

    dvec = x_lens + y_lens - 1
    usel = u_iota == y_lens
    alpha0 = jnp.where(u_iota == 0, 0.0, NEG).astype(f32)
    acc0 = jnp.zeros((N, U1), dtype=f32)
    negcol = jnp.full((N, 1), NEG, dtype=f32)

    def body(d, carry):
        alpha, acc = carry
        sb_d = sb_ref[d]
        sl_d = sl_ref[d]
        t_blank = alpha + sb_d
        hit = jnp.logical_and(dvec == d, usel)
        acc = acc + jnp.where(hit, t_blank, 0.0)
        t_label = alpha + sl_d
        t_label = jnp.concatenate([negcol, t_label[:, :-1]], axis=1)
        alpha = _logaddexp(t_blank, t_label)
        return alpha, acc

    _, acc = lax.fori_loop(0, NDIAG, body, (alpha0, acc0))
    out_ref[...] = (-jnp.sum(acc)).reshape(1, 1)


@jax.jit
def kernel(x, x_lens, y, y_lens, W_enc, b_enc, emb, W_am, b_am, W_lm, b_lm):
    y32 = y.astype(jnp.int32)
    zcol = jnp.zeros((N, 1), dtype=jnp.int32)
    sos = jnp.concatenate([zcol, y32], axis=1).reshape(N, U1, 1)
    ypad = jnp.concatenate([y32, zcol], axis=1).reshape(N, U1, 1)
    out = pl.pallas_call(
        _rnnt_kernel,
        out_shape=jax.ShapeDtypeStruct((1, 1), jnp.float32),
        scratch_shapes=[pltpu.VMEM((NDIAG, N, U1), jnp.float32),
                        pltpu.VMEM((NDIAG, N, U1), jnp.float32)],
    )(x.reshape(N * T, F), x_lens.astype(jnp.int32).reshape(N, 1),
      y_lens.astype(jnp.int32).reshape(N, 1), sos, ypad,
      W_enc, b_enc.reshape(1, D_ENC), emb, W_am, b_am.reshape(1, V),
      W_lm, b_lm.reshape(1, V))
    return out[0, 0]
